# baseline (speedup 1.0000x reference)
import functools

import jax
import jax.numpy as jnp
from jax import lax
from jax.experimental import pallas as pl
from jax.experimental.pallas import tpu as pltpu
from jax.experimental.pallas import tpu_sc as plsc

_NUM_CLASSES = 1000
_LENGTH = 256
_DIM_FULL = 150
_D = _LENGTH * _DIM_FULL
_B = 1024


@functools.cache
def _build():
    info = plsc.get_sparse_core_info()
    nc, ns = info.num_cores, info.num_subcores
    nw = nc * ns
    bpw = _B // nw

    mesh = plsc.VectorSubcoreMesh(core_axis_name="c", subcore_axis_name="s")

    def body(w_hbm, cls_hbm, out_hbm, idx_v, buf, gsem):
        wid = lax.axis_index("s") * nc + lax.axis_index("c")
        base = wid * bpw
        pltpu.sync_copy(cls_hbm.at[pl.ds(base, bpw)], idx_v)

        @pl.loop(0, bpw)
        def _row(i):
            pltpu.async_copy(w_hbm.at[idx_v.at[i]], buf, gsem).wait()
            pltpu.sync_copy(buf, out_hbm.at[pl.ds(base + i, 1)])

    return pl.kernel(
        body,
        out_type=jax.ShapeDtypeStruct((_B, _D), jnp.float32),
        mesh=mesh,
        scratch_types=[
            pltpu.VMEM((bpw, 1), jnp.int32),
            pltpu.VMEM((1, _D), jnp.float32),
            pltpu.SemaphoreType.DMA,
        ],
    )


def kernel(classes, W):
    w2 = W.reshape(_NUM_CLASSES, _D)
    cls2 = classes.astype(jnp.int32).reshape(_B, 1)
    out = _build()(w2, cls2)
    return out.reshape(_B, _LENGTH, _DIM_FULL)


# speedup vs baseline: 1.0194x; 1.0194x over previous
import functools

import jax
import jax.numpy as jnp
from jax import lax
from jax.experimental import pallas as pl
from jax.experimental.pallas import tpu as pltpu
from jax.experimental.pallas import tpu_sc as plsc

_NUM_CLASSES = 1000
_LENGTH = 256
_DIM_FULL = 150
_D = _LENGTH * _DIM_FULL
_B = 1024


@functools.cache
def _build():
    info = plsc.get_sparse_core_info()
    nc, ns = info.num_cores, info.num_subcores
    nw = nc * ns
    bpw = _B // nw

    mesh = plsc.VectorSubcoreMesh(core_axis_name="c", subcore_axis_name="s")

    nbuf = 3

    def body(w_hbm, cls_hbm, out_hbm, idx_v, buf, s0, s1, s2):
        sems = (s0, s1, s2)
        wid = lax.axis_index("s") * nc + lax.axis_index("c")
        base = wid * bpw
        pltpu.sync_copy(cls_hbm.at[pl.ds(base, bpw)], idx_v)

        ghandle = [None] * bpw
        ohandle = [None] * bpw

        def gather(i):
            s = i % nbuf
            ghandle[i] = pltpu.async_copy(
                w_hbm.at[idx_v.at[i]], buf.at[s], sems[s])

        def put(i):
            s = i % nbuf
            ohandle[i] = pltpu.async_copy(
                buf.at[s], out_hbm.at[pl.ds(base + i, 1)], sems[s])

        gather(0)
        for i in range(bpw):
            m = i + 1
            if m < bpw:
                if m >= nbuf:
                    ohandle[m - nbuf].wait()
                gather(m)
            ghandle[i].wait()
            put(i)
        for i in range(bpw - nbuf, bpw):
            ohandle[i].wait()

    return pl.kernel(
        body,
        out_type=jax.ShapeDtypeStruct((_B, _D), jnp.float32),
        mesh=mesh,
        scratch_types=[
            pltpu.VMEM((bpw, 1), jnp.int32),
            pltpu.VMEM((nbuf, 1, _D), jnp.float32),
            pltpu.SemaphoreType.DMA,
            pltpu.SemaphoreType.DMA,
            pltpu.SemaphoreType.DMA,
        ],
    )


def kernel(classes, W):
    w2 = W.reshape(_NUM_CLASSES, _D)
    cls2 = classes.astype(jnp.int32).reshape(_B, 1)
    out = _build()(w2, cls2)
    return out.reshape(_B, _LENGTH, _DIM_FULL)


# speedup vs baseline: 7.7989x; 7.6508x over previous
import functools

import jax
import jax.numpy as jnp
from jax import lax
from jax.experimental import pallas as pl
from jax.experimental.pallas import tpu as pltpu
from jax.experimental.pallas import tpu_sc as plsc

_C = 1000
_L = 256
_DF = 150
_B = 1024
_WROWS = _DF * _C
_OROWS = _DF * _B


@functools.cache
def _build():
    info = plsc.get_sparse_core_info()
    nc, ns = info.num_cores, info.num_subcores
    nw = nc * ns
    mw = _OROWS // nw
    kk = 80
    nch = mw // kk
    nbuf = 3

    mesh = plsc.VectorSubcoreMesh(core_axis_name="c", subcore_axis_name="s")

    def body(w_hbm, cls_hbm, out_hbm, cls_v, idx_buf, bufs, s0, s1, s2):
        sems = (s0, s1, s2)
        w2 = w_hbm.reshape(_WROWS, _L)
        out2 = out_hbm.reshape(_OROWS, _L)
        wid = lax.axis_index("s") * nc + lax.axis_index("c")
        wbase = wid * mw

        pltpu.sync_copy(cls_hbm, cls_v)

        @pl.loop(0, nch)
        def _idx(ch):
            row = idx_buf.at[ch]
            n0 = wbase + ch * kk
            for k in range(kk // 16):
                nk = n0 + 16 * k
                d = nk >> 10
                row[pl.ds(16 * k, 16)] = (
                    cls_v[pl.ds(nk & 1023, 16)] + d * 1000)

        def wait_slot(s):
            pltpu.make_async_copy(
                bufs.at[s], out2.at[pl.ds(0, kk)], sems[s]).wait()

        @pl.loop(0, nch // nbuf)
        def _go(g):
            for s in range(nbuf):
                c3 = g * nbuf + s

                @pl.when(g >= 1)
                def _drain(s=s):
                    wait_slot(s)

                pltpu.async_copy(w2.at[idx_buf.at[c3]], bufs.at[s], sems[s])
            for s in range(nbuf):
                c3 = g * nbuf + s
                wait_slot(s)
                pltpu.async_copy(
                    bufs.at[s], out2.at[pl.ds(wbase + c3 * kk, kk)], sems[s])

        for s in range(nbuf):
            wait_slot(s)

    return pl.kernel(
        body,
        out_type=jax.ShapeDtypeStruct((_DF, _B, _L), jnp.float32),
        mesh=mesh,
        scratch_types=[
            pltpu.VMEM((_B,), jnp.int32),
            pltpu.VMEM((nch, kk), jnp.int32),
            pltpu.VMEM((nbuf, kk, _L), jnp.float32),
            pltpu.SemaphoreType.DMA,
            pltpu.SemaphoreType.DMA,
            pltpu.SemaphoreType.DMA,
        ],
    )


def kernel(classes, W):
    w_t = jnp.transpose(W, (2, 0, 1))
    cls = classes.astype(jnp.int32)
    out_t = _build()(w_t, cls)
    return jnp.transpose(out_t, (1, 2, 0))


# speedup vs baseline: 7.9824x; 1.0235x over previous
import functools

import jax
import jax.numpy as jnp
from jax import lax
from jax.experimental import pallas as pl
from jax.experimental.pallas import tpu as pltpu
from jax.experimental.pallas import tpu_sc as plsc

_C = 1000
_L = 256
_DF = 150
_B = 1024
_WROWS = _DF * _C
_OROWS = _DF * _B


@functools.cache
def _build():
    info = plsc.get_sparse_core_info()
    nc, ns = info.num_cores, info.num_subcores
    nw = nc * ns
    mw = _OROWS // nw
    kk = 80
    nch = mw // kk
    nbuf = 4
    lag = 2
    ng = nch // nbuf

    mesh = plsc.VectorSubcoreMesh(core_axis_name="c", subcore_axis_name="s")

    def body(w_hbm, cls_hbm, out_hbm, cls_v, idx_buf, bufs, s0, s1, s2, s3):
        sems = (s0, s1, s2, s3)
        w2 = w_hbm.reshape(_WROWS, _L)
        out2 = out_hbm.reshape(_OROWS, _L)
        wid = lax.axis_index("s") * nc + lax.axis_index("c")
        wbase = wid * mw

        pltpu.sync_copy(cls_hbm, cls_v)

        def fill_idx(ch):
            row = idx_buf.at[ch]
            n0 = wbase + ch * kk
            for k in range(kk // 16):
                nk = n0 + 16 * k
                d = nk >> 10
                row[pl.ds(16 * k, 16)] = (
                    cls_v[pl.ds(nk & 1023, 16)] + d * 1000)

        def wait_slot(s):
            pltpu.make_async_copy(
                bufs.at[s], out2.at[pl.ds(0, kk)], sems[s]).wait()

        def issue_gather(c, s):
            pltpu.async_copy(w2.at[idx_buf.at[c]], bufs.at[s], sems[s])

        def issue_out(c, s):
            pltpu.async_copy(
                bufs.at[s], out2.at[pl.ds(wbase + c * kk, kk)], sems[s])

        for s in range(nbuf):
            fill_idx(s)
        for s in range(nbuf):
            issue_gather(s, s)
        for s in range(nbuf):
            fill_idx(nbuf + s)
        for s in range(lag):
            wait_slot(s)
            issue_out(s, s)

        @pl.loop(0, ng - 1)
        def _go(h):
            g = h + 1
            for s in range(nbuf):
                c = g * nbuf + s
                wait_slot(s)
                issue_gather(c, s)
                s2 = (s - lag) % nbuf
                wait_slot(s2)
                issue_out(c - lag, s2)

            @pl.when(h < ng - 2)
            def _prep():
                for s in range(nbuf):
                    fill_idx((g + 1) * nbuf + s)

        for s in range(nbuf - lag, nbuf):
            wait_slot(s)
            issue_out((ng - 1) * nbuf + s, s)
        for s in range(nbuf):
            wait_slot(s)

    return pl.kernel(
        body,
        out_type=jax.ShapeDtypeStruct((_DF, _B, _L), jnp.float32),
        mesh=mesh,
        scratch_types=[
            pltpu.VMEM((_B,), jnp.int32),
            pltpu.VMEM((nch, kk), jnp.int32),
            pltpu.VMEM((nbuf, kk, _L), jnp.float32),
            pltpu.SemaphoreType.DMA,
            pltpu.SemaphoreType.DMA,
            pltpu.SemaphoreType.DMA,
            pltpu.SemaphoreType.DMA,
        ],
    )


def kernel(classes, W):
    w_t = jnp.transpose(W, (2, 0, 1))
    cls = classes.astype(jnp.int32)
    out_t = _build()(w_t, cls)
    return jnp.transpose(out_t, (1, 2, 0))


# speedup vs baseline: 8.0304x; 1.0060x over previous
import functools

import jax
import jax.numpy as jnp
from jax import lax
from jax.experimental import pallas as pl
from jax.experimental.pallas import tpu as pltpu
from jax.experimental.pallas import tpu_sc as plsc

_C = 1000
_L = 256
_DF = 150
_B = 1024
_WROWS = _DF * _C
_OROWS = _DF * _B


@functools.cache
def _build():
    info = plsc.get_sparse_core_info()
    nc, ns = info.num_cores, info.num_subcores
    nw = nc * ns
    mw = _OROWS // nw
    kk = 96
    nch = mw // kk
    nbuf = 5
    lag = 2
    ng = nch // nbuf

    mesh = plsc.VectorSubcoreMesh(core_axis_name="c", subcore_axis_name="s")

    def body(w_hbm, cls_hbm, out_hbm, cls_v, idx_buf, bufs,
             s0, s1, s2, s3, s4):
        sems = (s0, s1, s2, s3, s4)
        w2 = w_hbm.reshape(_WROWS, _L)
        out2 = out_hbm.reshape(_OROWS, _L)
        wid = lax.axis_index("s") * nc + lax.axis_index("c")
        wbase = wid * mw

        pltpu.sync_copy(cls_hbm, cls_v)

        def fill_idx(ch):
            row = idx_buf.at[ch]
            n0 = wbase + ch * kk
            for k in range(kk // 16):
                nk = n0 + 16 * k
                d = nk >> 10
                row[pl.ds(16 * k, 16)] = (
                    cls_v[pl.ds(nk & 1023, 16)] + d * 1000)

        def wait_slot(s):
            pltpu.make_async_copy(
                bufs.at[s], out2.at[pl.ds(0, kk)], sems[s]).wait()

        def issue_gather(c, s):
            pltpu.async_copy(w2.at[idx_buf.at[c]], bufs.at[s], sems[s])

        def issue_out(c, s):
            pltpu.async_copy(
                bufs.at[s], out2.at[pl.ds(wbase + c * kk, kk)], sems[s])

        for s in range(nbuf):
            fill_idx(s)
        for s in range(nbuf):
            issue_gather(s, s)
        for s in range(nbuf):
            fill_idx(nbuf + s)
        for s in range(lag):
            wait_slot(s)
            issue_out(s, s)

        @pl.loop(0, ng - 1)
        def _go(h):
            g = h + 1
            for s in range(nbuf):
                c = g * nbuf + s
                wait_slot(s)
                issue_gather(c, s)
                s2 = (s - lag) % nbuf
                wait_slot(s2)
                issue_out(c - lag, s2)

            @pl.when(h < ng - 2)
            def _prep():
                for s in range(nbuf):
                    fill_idx((g + 1) * nbuf + s)

        for s in range(nbuf - lag, nbuf):
            wait_slot(s)
            issue_out((ng - 1) * nbuf + s, s)
        for s in range(nbuf):
            wait_slot(s)

    return pl.kernel(
        body,
        out_type=jax.ShapeDtypeStruct((_DF, _B, _L), jnp.float32),
        mesh=mesh,
        scratch_types=[
            pltpu.VMEM((_B,), jnp.int32),
            pltpu.VMEM((nch, kk), jnp.int32),
            pltpu.VMEM((nbuf, kk, _L), jnp.float32),
            pltpu.SemaphoreType.DMA,
            pltpu.SemaphoreType.DMA,
            pltpu.SemaphoreType.DMA,
            pltpu.SemaphoreType.DMA,
            pltpu.SemaphoreType.DMA,
        ],
    )


def kernel(classes, W):
    w_t = jnp.transpose(W, (2, 0, 1))
    cls = classes.astype(jnp.int32)
    out_t = _build()(w_t, cls)
    return jnp.transpose(out_t, (1, 2, 0))
